# Initial kernel scaffold; baseline (speedup 1.0000x reference)
import functools

import jax
import jax.numpy as jnp
from jax import lax
from jax.experimental import pallas as pl
from jax.experimental.pallas import tpu as pltpu
from jax.experimental.pallas import tpu_sc as plsc

N_IN = 16384
N_OUT = 16384
BATCH = 1024
DEGREE = 2

NC, NS, L = 2, 16, 16
NW = NC * NS
ROWS_PER_W = BATCH // NW
NVEC = N_OUT // L

DX = 1.0 / (N_IN - 1)
EPSILON = 1e-8
SCALE = 1.0 / (DX + EPSILON)

_mesh = plsc.VectorSubcoreMesh(core_axis_name="c", subcore_axis_name="s")


@functools.partial(
    pl.kernel,
    out_type=jax.ShapeDtypeStruct((BATCH, N_OUT), jnp.float32),
    mesh=_mesh,
    scratch_types=[
        pltpu.VMEM((2 * N_IN,), jnp.float32),
        pltpu.VMEM((N_OUT,), jnp.float32),
        pltpu.VMEM((N_OUT,), jnp.int32),
        pltpu.VMEM((N_OUT,), jnp.float32),
        pltpu.VMEM((N_OUT,), jnp.int32),
        pltpu.VMEM((N_OUT,), jnp.float32),
    ],
)
def _abacus_sc(act_hbm, sp_hbm, out_hbm, buf, orow, idx0, w0, idx1, w1):
    wid = lax.axis_index("c") * NS + lax.axis_index("s")
    iota2 = lax.iota(jnp.int32, L) * 2

    pltpu.sync_copy(sp_hbm, buf)

    def make_table(d, idx_t, w_t):
        @plsc.parallel_loop(0, NVEC, unroll=4)
        def _(j):
            q = iota2 + (j * (2 * L) + d)
            v = plsc.load_gather(buf, [q])
            v = jnp.clip(v, 0.0, 1.0)
            fi = (v * float(N_IN - 1)).astype(jnp.int32)
            fi = jnp.minimum(fi, N_IN - 2)
            xl = fi.astype(jnp.float32) * DX
            idx_t[pl.ds(j * L, L)] = fi
            w_t[pl.ds(j * L, L)] = (v - xl) * SCALE

    make_table(0, idx0, w0)
    make_table(1, idx1, w1)

    def do_row(r, carry):
        row = wid * ROWS_PER_W + r
        pltpu.sync_copy(act_hbm.at[row], buf.at[pl.ds(0, N_IN)])

        @plsc.parallel_loop(0, NVEC, unroll=4)
        def _(j):
            o = j * L
            i0 = idx0[pl.ds(o, L)]
            a0 = w0[pl.ds(o, L)]
            i1 = idx1[pl.ds(o, L)]
            a1 = w1[pl.ds(o, L)]
            y0l = plsc.load_gather(buf, [i0])
            y0r = plsc.load_gather(buf, [i0 + 1])
            y1l = plsc.load_gather(buf, [i1])
            y1r = plsc.load_gather(buf, [i1 + 1])
            t0 = y0l + (y0r - y0l) * a0
            t1 = y1l + (y1r - y1l) * a1
            orow[pl.ds(o, L)] = (1.0 - t0) * (1.0 - t1)

        pltpu.sync_copy(orow, out_hbm.at[row])
        return carry

    lax.fori_loop(0, ROWS_PER_W, do_row, 0)


def kernel(activations, sample_points):
    sp_flat = sample_points.reshape(-1)
    return _abacus_sc(activations, sp_flat)


# speedup vs baseline: 86.2630x; 86.2630x over previous
import functools

import jax
import jax.numpy as jnp
from jax import lax
from jax.experimental import pallas as pl
from jax.experimental.pallas import tpu as pltpu
from jax.experimental.pallas import tpu_sc as plsc

N_IN = 16384
N_OUT = 16384
BATCH = 1024
DEGREE = 2

NC, NS, L = 2, 16, 16
NW = NC * NS
ROWS_PER_W = BATCH // NW
NVEC = N_OUT // L

DX = 1.0 / (N_IN - 1)
EPSILON = 1e-8
SCALE = 1.0 / (DX + EPSILON)

_mesh = plsc.VectorSubcoreMesh(core_axis_name="c", subcore_axis_name="s")


@functools.partial(
    pl.kernel,
    out_type=jax.ShapeDtypeStruct((BATCH, N_OUT), jnp.float32),
    mesh=_mesh,
    compiler_params=pltpu.CompilerParams(needs_layout_passes=False),
    scratch_types=[
        pltpu.VMEM((2 * N_IN,), jnp.float32),
        pltpu.VMEM((N_OUT,), jnp.float32),
        pltpu.VMEM((N_OUT,), jnp.int32),
        pltpu.VMEM((N_OUT,), jnp.float32),
        pltpu.VMEM((N_OUT,), jnp.int32),
        pltpu.VMEM((N_OUT,), jnp.float32),
    ],
)
def _abacus_sc(act_hbm, sp_hbm, out_hbm, buf, orow, idx0, w0, idx1, w1):
    wid = lax.axis_index("c") * NS + lax.axis_index("s")
    iota2 = lax.iota(jnp.int32, L) * 2

    pltpu.sync_copy(sp_hbm, buf)

    def make_table(d, idx_t, w_t):
        def _(j, carry):
            q = iota2 + (j * (2 * L) + d)
            v = plsc.load_gather(buf, [q])
            v = jnp.clip(v, 0.0, 1.0)
            fi = (v * float(N_IN - 1)).astype(jnp.int32)
            fi = jnp.minimum(fi, N_IN - 2)
            xl = fi.astype(jnp.float32) * DX
            idx_t[pl.ds(j * L, L)] = fi
            w_t[pl.ds(j * L, L)] = (v - xl) * SCALE
            return carry

        lax.fori_loop(0, NVEC, _, 0)

    make_table(0, idx0, w0)
    make_table(1, idx1, w1)

    def do_row(r, carry):
        row = wid * ROWS_PER_W + r
        pltpu.sync_copy(act_hbm.at[row], buf.at[pl.ds(0, N_IN)])

        def inner(j, c):
            o = j * L
            i0 = idx0[pl.ds(o, L)]
            a0 = w0[pl.ds(o, L)]
            i1 = idx1[pl.ds(o, L)]
            a1 = w1[pl.ds(o, L)]
            y0l = plsc.load_gather(buf, [i0])
            y0r = plsc.load_gather(buf, [i0 + 1])
            y1l = plsc.load_gather(buf, [i1])
            y1r = plsc.load_gather(buf, [i1 + 1])
            t0 = y0l + (y0r - y0l) * a0
            t1 = y1l + (y1r - y1l) * a1
            orow[pl.ds(o, L)] = (1.0 - t0) * (1.0 - t1)
            return c

        lax.fori_loop(0, NVEC, inner, 0)
        pltpu.sync_copy(orow, out_hbm.at[row])
        return carry

    lax.fori_loop(0, ROWS_PER_W, do_row, 0)


def kernel(activations, sample_points):
    sp_flat = sample_points.reshape(-1)
    return _abacus_sc(activations, sp_flat)
